# baseline (speedup 1.0000x reference)
import jax
import jax.numpy as jnp
from jax.experimental import pallas as pl
from jax.experimental.pallas import tpu as pltpu
from jax.experimental.pallas import tpu_sc as plsc

_LIFETIME_K = 5
_NEG = -3.0e38
_SC_LANES = 16
_SC_CORES = 2
_SC_SUBCORES = 16


def _make_iotas(w, h):
    fi_i = (jax.lax.broadcasted_iota(jnp.int32, (w, h, 1), 0) * h
            + jax.lax.broadcasted_iota(jnp.int32, (w, h, 1), 1))
    fi = fi_i.astype(jnp.float32)
    return fi, (w * h - 1.0) - fi


def _stage_a(x_ref, win_ref, i1_ref, i2_ref, fi_ref, rfi_ref):
    bb, w, h, c = x_ref.shape
    half = pl.num_programs(0) // 2
    step = pl.program_id(0)

    @pl.when((step == 0) | (step == half))
    def _():
        fi, rfi = _make_iotas(w, h)
        fi_ref[...] = fi
        rfi_ref[...] = rfi

    fi = fi_ref[...]
    rfi = rfi_ref[...]
    for j in range(bb):
        y = x_ref[j]
        m = jnp.max(jnp.max(y, axis=0), axis=0)
        eqf = (y == m[None, None, :]).astype(jnp.float32)
        i2 = jnp.max(jnp.max(eqf * fi, axis=0), axis=0)
        i1 = (w * h - 1.0) - jnp.max(jnp.max(eqf * rfi, axis=0), axis=0)
        win_ref[j, 0, :] = m
        i1_ref[j, 0, :] = i1
        i2_ref[j, 0, :] = i2


def _sc_lifetime_mask(w_hbm, m2_hbm, wv, wm, m2v, thr, mx, cnt, rem, done):
    slab_len = wv.shape[0]
    b = slab_len // _SC_LANES
    n_slabs = w_hbm.shape[0] // slab_len
    unit = jax.lax.axis_index("c") * _SC_SUBCORES + jax.lax.axis_index("s")

    @pl.when(unit < n_slabs)
    def _():
        pltpu.sync_copy(w_hbm.at[pl.ds(unit * slab_len, slab_len)], wv)

        thr[...] = jnp.full((_SC_LANES,), _NEG, jnp.float32)
        rem[...] = jnp.zeros((_SC_LANES,), jnp.float32)
        done[...] = jnp.zeros((_SC_LANES,), jnp.float32)

        unroll = 4
        for r in range(_LIFETIME_K):
            src = wv if r == 0 else wm
            mx[...] = jnp.full((_SC_LANES,), _NEG, jnp.float32)

            @pl.loop(0, b, step=unroll)
            def _(i, src=src):
                acc = mx[...]
                for u in range(unroll):
                    acc = jnp.maximum(
                        acc, src[pl.ds((i + u) * _SC_LANES, _SC_LANES)])
                mx[...] = acc

            cnt[...] = jnp.zeros((_SC_LANES,), jnp.float32)

            @pl.loop(0, b, step=unroll)
            def _(i, src=src):
                acc = cnt[...]
                for u in range(unroll):
                    w_i = src[pl.ds((i + u) * _SC_LANES, _SC_LANES)]
                    eq = w_i == mx[...]
                    acc = acc + jnp.where(eq, 1.0, 0.0)
                    wm[pl.ds((i + u) * _SC_LANES, _SC_LANES)] = jnp.where(
                        eq, _NEG, w_i)
                cnt[...] = acc

            trig = jnp.where(
                (rem[...] + cnt[...] >= float(_LIFETIME_K))
                & (done[...] < 0.5), 1.0, 0.0)
            thr[...] = jnp.where(trig > 0.5, mx[...], thr[...])
            done[...] = jnp.maximum(done[...], trig)
            rem[...] = rem[...] + cnt[...]

        @pl.loop(0, b, step=4)
        def _(i):
            for u in range(4):
                m2v[pl.ds((i + u) * _SC_LANES, _SC_LANES)] = jnp.where(
                    wv[pl.ds((i + u) * _SC_LANES, _SC_LANES)] >= thr[...],
                    1.0, 0.0)

        pltpu.sync_copy(m2v, m2_hbm.at[pl.ds(unit * slab_len, slab_len)])


def _stage_b(m2_full_ref, win_ref, i1_ref, i2_ref, out_ref, fi_ref):
    bb, w, h, c = out_ref.shape
    step = pl.program_id(0)
    half = pl.num_programs(0) // 2

    @pl.when((step == 0) | (step == half))
    def _():
        fi, _ = _make_iotas(w, h)
        fi_ref[...] = fi

    fi = fi_ref[...]
    for j in range(bb):
        b = step * bb + j
        wrow = win_ref[j, 0, :]
        i1 = i1_ref[j, 0, :]
        i2 = i2_ref[j, 0, :]
        m2row = m2_full_ref[pl.ds(b, 1), 0, :][0]
        keep = (fi == i1[None, None, :]) | (fi == i2[None, None, :])
        val = (wrow * m2row)[None, None, :]
        out_ref[j] = jnp.where(keep, val, 0.0)


def kernel(x):
    b, c, w, h = x.shape
    f32 = jnp.float32
    bb = 8
    grid = b // bb
    xt = jnp.transpose(x, (0, 2, 3, 1))
    winners, i1, i2 = pl.pallas_call(
        _stage_a,
        grid=(grid,),
        in_specs=[pl.BlockSpec((bb, w, h, c), lambda i: (i, 0, 0, 0))],
        out_specs=[pl.BlockSpec((bb, 1, c), lambda i: (i, 0, 0))] * 3,
        out_shape=[jax.ShapeDtypeStruct((b, 1, c), f32)] * 3,
        scratch_shapes=[pltpu.VMEM((w, h, 1), f32)] * 2,
        compiler_params=pltpu.CompilerParams(
            dimension_semantics=("parallel",)),
    )(xt)
    n_slabs = c // _SC_LANES
    slab_len = b * _SC_LANES
    ws = winners.reshape(b, n_slabs, _SC_LANES).transpose(1, 0, 2).reshape(-1)
    sc_mesh = plsc.VectorSubcoreMesh(core_axis_name="c", subcore_axis_name="s")
    m2_flat = pl.kernel(
        _sc_lifetime_mask,
        out_type=jax.ShapeDtypeStruct((n_slabs * slab_len,), f32),
        mesh=sc_mesh,
        scratch_types=[pltpu.VMEM((slab_len,), f32),
                       pltpu.VMEM((slab_len,), f32),
                       pltpu.VMEM((slab_len,), f32),
                       pltpu.VMEM((_SC_LANES,), f32),
                       pltpu.VMEM((_SC_LANES,), f32),
                       pltpu.VMEM((_SC_LANES,), f32),
                       pltpu.VMEM((_SC_LANES,), f32),
                       pltpu.VMEM((_SC_LANES,), f32)],
    )(ws)
    m2 = (m2_flat.reshape(n_slabs, b, _SC_LANES)
          .transpose(1, 0, 2).reshape(b, 1, c))
    out_t = pl.pallas_call(
        _stage_b,
        grid=(grid,),
        in_specs=[pl.BlockSpec((b, 1, c), lambda i: (0, 0, 0)),
                  pl.BlockSpec((bb, 1, c), lambda i: (i, 0, 0)),
                  pl.BlockSpec((bb, 1, c), lambda i: (i, 0, 0)),
                  pl.BlockSpec((bb, 1, c), lambda i: (i, 0, 0))],
        out_specs=pl.BlockSpec((bb, w, h, c), lambda i: (i, 0, 0, 0)),
        out_shape=jax.ShapeDtypeStruct((b, w, h, c), f32),
        scratch_shapes=[pltpu.VMEM((w, h, 1), f32)],
        compiler_params=pltpu.CompilerParams(
            dimension_semantics=("parallel",)),
    )(m2, winners, i1, i2)
    return jnp.transpose(out_t, (0, 3, 1, 2))


# speedup vs baseline: 1.0184x; 1.0184x over previous
import jax
import jax.numpy as jnp
from jax.experimental import pallas as pl
from jax.experimental.pallas import tpu as pltpu
from jax.experimental.pallas import tpu_sc as plsc

_LIFETIME_K = 5
_NEG = -3.0e38
_SC_LANES = 16
_SC_SUBCORES = 16


def _make_iotas(w, h):
    fi_i = (jax.lax.broadcasted_iota(jnp.int32, (w, h, 1), 0) * h
            + jax.lax.broadcasted_iota(jnp.int32, (w, h, 1), 1))
    fi = fi_i.astype(jnp.float32)
    return fi, (w * h - 1.0) - fi


def _stage_a(x_ref, win_ref, i1_ref, i2_ref, fi_ref, rfi_ref):
    bb, w, h, c = x_ref.shape
    half = pl.num_programs(0) // 2
    step = pl.program_id(0)

    @pl.when((step == 0) | (step == half))
    def _():
        fi, rfi = _make_iotas(w, h)
        fi_ref[...] = fi
        rfi_ref[...] = rfi

    fi = fi_ref[...]
    rfi = rfi_ref[...]
    for j in range(bb):
        y = x_ref[j]
        m = jnp.max(jnp.max(y, axis=0), axis=0)
        eqf = (y == m[None, None, :]).astype(jnp.float32)
        i2 = jnp.max(jnp.max(eqf * fi, axis=0), axis=0)
        i1 = (w * h - 1.0) - jnp.max(jnp.max(eqf * rfi, axis=0), axis=0)
        win_ref[j, 0, :] = m
        i1_ref[j, 0, :] = i1
        i2_ref[j, 0, :] = i2


def _sc_lifetime_mask(w_hbm, m2_hbm, wv, wm, m2v, thr, mx, cnt, rem, done):
    slab_len = wv.shape[0]
    b = slab_len // _SC_LANES
    n_slabs = w_hbm.shape[0] // slab_len
    unit = jax.lax.axis_index("c") * _SC_SUBCORES + jax.lax.axis_index("s")

    @pl.when(unit < n_slabs)
    def _():
        pltpu.sync_copy(w_hbm.at[pl.ds(unit * slab_len, slab_len)], wv)

        thr[...] = jnp.full((_SC_LANES,), _NEG, jnp.float32)
        rem[...] = jnp.zeros((_SC_LANES,), jnp.float32)
        done[...] = jnp.zeros((_SC_LANES,), jnp.float32)

        unroll = 4
        for r in range(_LIFETIME_K):
            src = wv if r == 0 else wm
            mx[...] = jnp.full((_SC_LANES,), _NEG, jnp.float32)

            @pl.loop(0, b, step=unroll)
            def _(i, src=src):
                acc = mx[...]
                for u in range(unroll):
                    acc = jnp.maximum(
                        acc, src[pl.ds((i + u) * _SC_LANES, _SC_LANES)])
                mx[...] = acc

            cnt[...] = jnp.zeros((_SC_LANES,), jnp.float32)

            @pl.loop(0, b, step=unroll)
            def _(i, src=src):
                acc = cnt[...]
                for u in range(unroll):
                    w_i = src[pl.ds((i + u) * _SC_LANES, _SC_LANES)]
                    eq = w_i == mx[...]
                    acc = acc + jnp.where(eq, 1.0, 0.0)
                    wm[pl.ds((i + u) * _SC_LANES, _SC_LANES)] = jnp.where(
                        eq, _NEG, w_i)
                cnt[...] = acc

            trig = jnp.where(
                (rem[...] + cnt[...] >= float(_LIFETIME_K))
                & (done[...] < 0.5), 1.0, 0.0)
            thr[...] = jnp.where(trig > 0.5, mx[...], thr[...])
            done[...] = jnp.maximum(done[...], trig)
            rem[...] = rem[...] + cnt[...]

        @pl.loop(0, b, step=4)
        def _(i):
            for u in range(4):
                m2v[pl.ds((i + u) * _SC_LANES, _SC_LANES)] = jnp.where(
                    wv[pl.ds((i + u) * _SC_LANES, _SC_LANES)] >= thr[...],
                    1.0, 0.0)

        pltpu.sync_copy(m2v, m2_hbm.at[pl.ds(unit * slab_len, slab_len)])


def _stage_b(m2_full_ref, win_ref, i1_ref, i2_ref, out_ref, fi_ref):
    bb, w, h, c = out_ref.shape
    step = pl.program_id(0)
    half = pl.num_programs(0) // 2

    @pl.when((step == 0) | (step == half))
    def _():
        fi, _ = _make_iotas(w, h)
        fi_ref[...] = fi

    fi = fi_ref[...]
    for j in range(bb):
        b = step * bb + j
        wrow = win_ref[j, 0, :]
        i1 = i1_ref[j, 0, :]
        i2 = i2_ref[j, 0, :]
        m2row = m2_full_ref[pl.ds(b, 1), 0, :][0]
        keep = (fi == i1[None, None, :]) | (fi == i2[None, None, :])
        val = (wrow * m2row)[None, None, :]
        out_ref[j] = jnp.where(keep, val, 0.0)


def kernel(x):
    b, c, w, h = x.shape
    f32 = jnp.float32
    bb = 8
    grid = b // bb
    xt = jnp.transpose(x, (0, 2, 3, 1))
    winners, i1, i2 = pl.pallas_call(
        _stage_a,
        grid=(grid,),
        in_specs=[pl.BlockSpec((bb, w, h, c), lambda i: (i, 0, 0, 0))],
        out_specs=[pl.BlockSpec((bb, 1, c), lambda i: (i, 0, 0))] * 3,
        out_shape=[jax.ShapeDtypeStruct((b, 1, c), f32)] * 3,
        scratch_shapes=[pltpu.VMEM((w, h, 1), f32)] * 2,
        compiler_params=pltpu.CompilerParams(
            dimension_semantics=("parallel",)),
    )(xt)
    n_slabs = c // _SC_LANES
    slab_len = b * _SC_LANES
    ws = winners.reshape(b, n_slabs, _SC_LANES).transpose(1, 0, 2).reshape(-1)
    sc_mesh = plsc.VectorSubcoreMesh(core_axis_name="c", subcore_axis_name="s")
    m2_flat = pl.kernel(
        _sc_lifetime_mask,
        out_type=jax.ShapeDtypeStruct((n_slabs * slab_len,), f32),
        mesh=sc_mesh,
        scratch_types=[pltpu.VMEM((slab_len,), f32),
                       pltpu.VMEM((slab_len,), f32),
                       pltpu.VMEM((slab_len,), f32),
                       pltpu.VMEM((_SC_LANES,), f32),
                       pltpu.VMEM((_SC_LANES,), f32),
                       pltpu.VMEM((_SC_LANES,), f32),
                       pltpu.VMEM((_SC_LANES,), f32),
                       pltpu.VMEM((_SC_LANES,), f32)],
    )(ws)
    m2 = (m2_flat.reshape(n_slabs, b, _SC_LANES)
          .transpose(1, 0, 2).reshape(b, 1, c))
    out_t = pl.pallas_call(
        _stage_b,
        grid=(grid,),
        in_specs=[pl.BlockSpec((b, 1, c), lambda i: (0, 0, 0)),
                  pl.BlockSpec((bb, 1, c), lambda i: (i, 0, 0)),
                  pl.BlockSpec((bb, 1, c), lambda i: (i, 0, 0)),
                  pl.BlockSpec((bb, 1, c), lambda i: (i, 0, 0))],
        out_specs=pl.BlockSpec((bb, w, h, c), lambda i: (i, 0, 0, 0)),
        out_shape=jax.ShapeDtypeStruct((b, w, h, c), f32),
        scratch_shapes=[pltpu.VMEM((w, h, 1), f32)],
        compiler_params=pltpu.CompilerParams(
            dimension_semantics=("parallel",)),
    )(m2, winners, i1, i2)
    return jnp.transpose(out_t, (0, 3, 1, 2))
